# baseline (speedup 1.0000x reference)
import math

import jax
import jax.numpy as jnp
from jax import lax
from jax.experimental import pallas as pl
from jax.experimental.pallas import tpu as pltpu
from jax.experimental.pallas import tpu_sc as plsc

D_MODEL = 128
SCALE = math.sqrt(float(D_MODEL))

_INFO = plsc.get_sparse_core_info()
NC = _INFO.num_cores
NS = _INFO.num_subcores
NW = NC * NS
LANES = _INFO.num_lanes

GROWS = 128
G = 1
CHUNK = G * GROWS
NBUF = 5
DEPTH = 3


def _emb_body(x_hbm, table_hbm, out_hbm, idx_v, *scratch):
    bufs = scratch[:NBUF]
    gsem = scratch[NBUF:2 * NBUF]
    osem = scratch[2 * NBUF:]

    n_chunks = x_hbm.shape[1] // G
    wid = lax.axis_index("s") * NC + lax.axis_index("c")
    base = wid * (n_chunks * CHUNK)

    pltpu.sync_copy(x_hbm.at[wid], idx_v)

    def fire_gather(j, b):
        for k in range(G):
            pltpu.async_copy(table_hbm.at[idx_v.at[G * j + k]],
                             bufs[b].at[pl.ds(k * GROWS, GROWS)], gsem[b])

    def wait_gather(j, b):
        for k in range(G):
            pltpu.make_async_copy(table_hbm.at[idx_v.at[G * j + k]],
                                  bufs[b].at[pl.ds(k * GROWS, GROWS)],
                                  gsem[b]).wait()

    def scale_chunk(buf):
        @plsc.parallel_loop(0, CHUNK, step=1, unroll=4)
        def _(r):
            for c in range(D_MODEL // LANES):
                sl = pl.ds(c * LANES, LANES)
                buf[r, sl] = buf[r, sl] * SCALE

    def out_slice(j):
        return out_hbm.at[pl.ds(base + j * CHUNK, CHUNK)]

    for b in range(DEPTH):
        fire_gather(b, b)

    def chunk_step(j, b):
        bp = (b + DEPTH) % NBUF

        @pl.when(j + DEPTH < n_chunks)
        def _():
            @pl.when(j >= NBUF - DEPTH)
            def _():
                jprev = j - (NBUF - DEPTH)
                pltpu.make_async_copy(bufs[bp], out_slice(jprev),
                                      osem[bp]).wait()
            fire_gather(j + DEPTH, bp)

        wait_gather(j, b)
        scale_chunk(bufs[b])
        pltpu.async_copy(bufs[b], out_slice(j), osem[b])

    n_groups = n_chunks // NBUF
    if n_groups > 0:
        def group(g, _):
            for b in range(NBUF):
                chunk_step(g * NBUF + b, b)
            return 0
        lax.fori_loop(0, n_groups, group, 0)

    for j in range(n_groups * NBUF, n_chunks):
        chunk_step(j, j % NBUF)

    for i in range(NBUF):
        j = n_chunks - NBUF + i
        pltpu.make_async_copy(bufs[j % NBUF], out_slice(j),
                              osem[j % NBUF]).wait()


def kernel(x, table):
    bsz, seq = x.shape
    total = bsz * seq
    assert total % (NW * CHUNK) == 0
    n_chunks = total // (NW * CHUNK)

    xw = x.reshape(NW, n_chunks * G, GROWS).astype(jnp.int32)

    mesh = plsc.VectorSubcoreMesh(core_axis_name="c", subcore_axis_name="s")
    run = pl.kernel(
        _emb_body,
        out_type=jax.ShapeDtypeStruct((total, D_MODEL), jnp.float32),
        mesh=mesh,
        scratch_types=(
            [pltpu.VMEM((n_chunks * G, GROWS), jnp.int32)]
            + [pltpu.VMEM((CHUNK, D_MODEL), jnp.float32)] * NBUF
            + [pltpu.SemaphoreType.DMA] * (2 * NBUF)
        ),
    )
    out = run(xw, table)
    return out.reshape(bsz, seq, D_MODEL)


# speedup vs baseline: 1.0289x; 1.0289x over previous
import math

import jax
import jax.numpy as jnp
from jax import lax
from jax.experimental import pallas as pl
from jax.experimental.pallas import tpu as pltpu
from jax.experimental.pallas import tpu_sc as plsc

D_MODEL = 128
SCALE = math.sqrt(float(D_MODEL))

_INFO = plsc.get_sparse_core_info()
NC = _INFO.num_cores
NS = _INFO.num_subcores
NW = NC * NS
LANES = _INFO.num_lanes

GROWS = 128
G = 1
CHUNK = G * GROWS
NBUF = 7
DEPTH = 5


def _emb_body(x_hbm, table_hbm, out_hbm, idx_v, *scratch):
    bufs = scratch[:NBUF]
    gsem = scratch[NBUF:2 * NBUF]
    osem = scratch[2 * NBUF:]

    n_chunks = x_hbm.shape[1] // G
    wid = lax.axis_index("s") * NC + lax.axis_index("c")
    base = wid * (n_chunks * CHUNK)

    pltpu.sync_copy(x_hbm.at[wid], idx_v)

    def fire_gather(j, b):
        for k in range(G):
            pltpu.async_copy(table_hbm.at[idx_v.at[G * j + k]],
                             bufs[b].at[pl.ds(k * GROWS, GROWS)], gsem[b])

    def wait_gather(j, b):
        for k in range(G):
            pltpu.make_async_copy(table_hbm.at[idx_v.at[G * j + k]],
                                  bufs[b].at[pl.ds(k * GROWS, GROWS)],
                                  gsem[b]).wait()

    def scale_chunk(buf):
        @plsc.parallel_loop(0, CHUNK, step=1, unroll=4)
        def _(r):
            for c in range(D_MODEL // LANES):
                sl = pl.ds(c * LANES, LANES)
                buf[r, sl] = buf[r, sl] * SCALE

    def out_slice(j):
        return out_hbm.at[pl.ds(base + j * CHUNK, CHUNK)]

    for b in range(DEPTH):
        fire_gather(b, b)

    def chunk_step(j, b):
        bp = (b + DEPTH) % NBUF

        @pl.when(j + DEPTH < n_chunks)
        def _():
            @pl.when(j >= NBUF - DEPTH)
            def _():
                jprev = j - (NBUF - DEPTH)
                pltpu.make_async_copy(bufs[bp], out_slice(jprev),
                                      osem[bp]).wait()
            fire_gather(j + DEPTH, bp)

        wait_gather(j, b)
        scale_chunk(bufs[b])
        pltpu.async_copy(bufs[b], out_slice(j), osem[b])

    n_groups = n_chunks // NBUF
    if n_groups > 0:
        def group(g, _):
            for b in range(NBUF):
                chunk_step(g * NBUF + b, b)
            return 0
        lax.fori_loop(0, n_groups, group, 0)

    for j in range(n_groups * NBUF, n_chunks):
        chunk_step(j, j % NBUF)

    for i in range(NBUF):
        j = n_chunks - NBUF + i
        pltpu.make_async_copy(bufs[j % NBUF], out_slice(j),
                              osem[j % NBUF]).wait()


def kernel(x, table):
    bsz, seq = x.shape
    total = bsz * seq
    assert total % (NW * CHUNK) == 0
    n_chunks = total // (NW * CHUNK)

    xw = x.reshape(NW, n_chunks * G, GROWS).astype(jnp.int32)

    mesh = plsc.VectorSubcoreMesh(core_axis_name="c", subcore_axis_name="s")
    run = pl.kernel(
        _emb_body,
        out_type=jax.ShapeDtypeStruct((total, D_MODEL), jnp.float32),
        mesh=mesh,
        scratch_types=(
            [pltpu.VMEM((n_chunks * G, GROWS), jnp.int32)]
            + [pltpu.VMEM((CHUNK, D_MODEL), jnp.float32)] * NBUF
            + [pltpu.SemaphoreType.DMA] * (2 * NBUF)
        ),
    )
    out = run(xw, table)
    return out.reshape(bsz, seq, D_MODEL)
